# Initial kernel scaffold; baseline (speedup 1.0000x reference)
import jax
import jax.numpy as jnp
from jax.experimental import pallas as pl


def kernel(x):
    raise NotImplementedError("write your pallas kernel here")


# speedup vs baseline: 10.2989x; 10.2989x over previous
import functools

import jax
import jax.numpy as jnp
from jax import lax
from jax.experimental import pallas as pl
from jax.experimental.pallas import tpu as pltpu

_K = 16
_ITERS = 10
_CHUNK = 8192


def _body(x_hbm, out_ref, pts, sem, *, C, N, K, iters, chunk):
    b = pl.program_id(0)
    cp = pltpu.make_async_copy(x_hbm.at[b], pts, sem)
    cp.start()
    cp.wait()

    cT0 = pts[:, :K]
    eye = (
        lax.broadcasted_iota(jnp.int32, (K, K), 0)
        == lax.broadcasted_iota(jnp.int32, (K, K), 1)
    ).astype(jnp.float32)
    ones_row = jnp.ones((1, chunk), jnp.float32)
    nchunks = N // chunk

    def one_iter(_, cT):
        gram = lax.dot_general(
            cT, cT, (((0,), (0,)), ((), ())),
            preferred_element_type=jnp.float32,
            precision=lax.Precision.HIGHEST,
        )
        cen_sq = jnp.sum(gram * eye, axis=1, keepdims=True)

        def chunk_body(ci, carry):
            sums, counts = carry
            pc = pts[:, pl.ds(ci * chunk, chunk)]
            s = lax.dot_general(
                cT.astype(jnp.bfloat16), pc.astype(jnp.bfloat16),
                (((0,), (0,)), ((), ())),
                preferred_element_type=jnp.float32,
            )
            score = cen_sq - 2.0 * s
            m = jnp.min(score, axis=0, keepdims=True)
            io = lax.broadcasted_iota(jnp.int32, (K, chunk), 0)
            idx = jnp.min(jnp.where(score <= m, io, K), axis=0, keepdims=True)
            onehot = (io == idx).astype(jnp.float32)
            sums = sums + lax.dot_general(
                pc, onehot, (((1,), (1,)), ((), ())),
                preferred_element_type=jnp.float32,
                precision=lax.Precision.HIGHEST,
            )
            counts = counts + lax.dot_general(
                ones_row, onehot, (((1,), (1,)), ((), ())),
                preferred_element_type=jnp.float32,
            )
            return sums, counts

        sums, counts = lax.fori_loop(
            0,
            nchunks,
            chunk_body,
            (jnp.zeros((C, K), jnp.float32), jnp.zeros((1, K), jnp.float32)),
        )
        newcT = sums / jnp.maximum(counts, 1.0)
        return jnp.where(counts > 0.0, newcT, cT)

    cT = lax.fori_loop(0, iters, one_iter, cT0)
    out_ref[...] = cT[None]


def kernel(x):
    B, C, H, W = x.shape
    N = H * W
    xr = x.reshape(B, C, N)
    body = functools.partial(
        _body, C=C, N=N, K=_K, iters=_ITERS, chunk=_CHUNK
    )
    out = pl.pallas_call(
        body,
        grid=(B,),
        in_specs=[pl.BlockSpec(memory_space=pl.ANY)],
        out_specs=pl.BlockSpec((1, C, _K), lambda b: (b, 0, 0)),
        out_shape=jax.ShapeDtypeStruct((B, C, _K), jnp.float32),
        scratch_shapes=[
            pltpu.VMEM((C, N), jnp.float32),
            pltpu.SemaphoreType.DMA,
        ],
    )(xr)
    return out[..., None]
